# Initial kernel scaffold; baseline (speedup 1.0000x reference)
import jax
import jax.numpy as jnp
from jax.experimental import pallas as pl


def kernel(tMP, tKF, measurements, idxMP, idxKF):
    raise NotImplementedError("write your pallas kernel here")


# speedup vs baseline: 27.7612x; 27.7612x over previous
import functools

import jax
import jax.numpy as jnp
from jax import lax
from jax.experimental import pallas as pl
from jax.experimental.pallas import tpu as pltpu
from jax.experimental.pallas import tpu_sc as plsc

N_KF = 16
N_MP = 32
M = 65536
FX = 320.0
FY = 320.0
CX = 320.0
CY = 240.0

NUM_WORKERS = 32
MEAS_PER_W = M // NUM_WORKERS
WORDS_PER_W = MEAS_PER_W * 4
N_TABLE = N_KF * N_MP
TABLE_WORDS = N_TABLE * 4


def _f64_words(val_f32):
    bu = plsc.bitcast(val_f32, jnp.uint32)
    m = bu & jnp.uint32(0x7FFFFF)
    e = lax.shift_right_logical(bu, jnp.uint32(23)) & jnp.uint32(0xFF)
    s = bu & jnp.uint32(0x80000000)
    hi = s | lax.shift_left(e + jnp.uint32(896), jnp.uint32(20)) | lax.shift_right_logical(m, jnp.uint32(3))
    lo = lax.shift_left(m, jnp.uint32(29))
    tiny = e == jnp.uint32(0)
    hi = jnp.where(tiny, s, hi)
    lo = jnp.where(tiny, jnp.uint32(0), lo)
    return plsc.bitcast(lo, jnp.int32), plsc.bitcast(hi, jnp.int32)


def _sc_body(meas_hbm, tkf_hbm, tmp_hbm, idxkf_hbm, idxmp_hbm, out_hbm,
             meas_v, out_v, table_v, tkf_v, tmp_v, idxkf_v, idxmp_v,
             invkf_v, invmp_v):
    wid = lax.axis_index("s") * 2 + lax.axis_index("c")
    iota = lax.iota(jnp.int32, 16)
    iota4 = lax.shift_left(iota, jnp.int32(2))

    pltpu.sync_copy(tkf_hbm, tkf_v)
    pltpu.sync_copy(tmp_hbm, tmp_v)
    pltpu.sync_copy(idxkf_hbm, idxkf_v)
    pltpu.sync_copy(idxmp_hbm, idxmp_v)
    pltpu.sync_copy(meas_hbm.at[pl.ds(wid * WORDS_PER_W, WORDS_PER_W)], meas_v)

    plsc.store_scatter(invkf_v, [idxkf_v[...]], iota)
    plsc.store_scatter(invmp_v, [idxmp_v[pl.ds(0, 16)]], iota)
    plsc.store_scatter(invmp_v, [idxmp_v[pl.ds(16, 16)]], iota + 16)

    def table_step(_, cbase):
        c = cbase + iota
        kid = lax.shift_right_logical(c, jnp.int32(5))
        mid = c & 31
        kpos = plsc.load_gather(invkf_v, [kid])
        mpos = plsc.load_gather(invmp_v, [mid])
        kbase = lax.shift_left(kpos, jnp.int32(3)) + lax.shift_left(kpos, jnp.int32(2))
        mbase = mpos + lax.shift_left(mpos, jnp.int32(1))
        r = [plsc.load_gather(tkf_v, [kbase + j]) for j in range(12)]
        px = plsc.load_gather(tmp_v, [mbase])
        py = plsc.load_gather(tmp_v, [mbase + 1])
        pz = plsc.load_gather(tmp_v, [mbase + 2])
        x = r[0] * px + r[1] * py + r[2] * pz + r[3]
        y = r[4] * px + r[5] * py + r[6] * pz + r[7]
        z = r[8] * px + r[9] * py + r[10] * pz + r[11]
        inv = jnp.float32(1.0) / z
        ptx = x * inv * jnp.float32(FX) + jnp.float32(CX)
        pty = y * inv * jnp.float32(FY) + jnp.float32(CY)
        xlo, xhi = _f64_words(ptx)
        ylo, yhi = _f64_words(pty)
        c4 = lax.shift_left(c, jnp.int32(2))
        plsc.store_scatter(table_v, [c4], xlo)
        plsc.store_scatter(table_v, [c4 + 1], xhi)
        plsc.store_scatter(table_v, [c4 + 2], ylo)
        plsc.store_scatter(table_v, [c4 + 3], yhi)
        return cbase + jnp.int32(16)

    lax.fori_loop(0, N_TABLE // 16, table_step, jnp.int32(0), unroll=4)

    def gather_step(_, base):
        widx = base + iota4
        kf = plsc.load_gather(meas_v, [widx])
        mp = plsc.load_gather(meas_v, [widx + 2])
        c4 = lax.shift_left(kf, jnp.int32(7)) + lax.shift_left(mp, jnp.int32(2))
        xlo = plsc.load_gather(table_v, [c4])
        xhi = plsc.load_gather(table_v, [c4 + 1])
        ylo = plsc.load_gather(table_v, [c4 + 2])
        yhi = plsc.load_gather(table_v, [c4 + 3])
        plsc.store_scatter(out_v, [widx], xlo)
        plsc.store_scatter(out_v, [widx + 1], xhi)
        plsc.store_scatter(out_v, [widx + 2], ylo)
        plsc.store_scatter(out_v, [widx + 3], yhi)
        return base + jnp.int32(64)

    lax.fori_loop(0, MEAS_PER_W // 16, gather_step, jnp.int32(0), unroll=4)

    pltpu.sync_copy(out_v, out_hbm.at[pl.ds(wid * WORDS_PER_W, WORDS_PER_W)])


def kernel(tMP, tKF, measurements, idxMP, idxKF):
    meas_words = lax.bitcast_convert_type(measurements, jnp.int32).reshape(M * 4)
    tkf32 = tKF.astype(jnp.float32)[:, :3, :].reshape(N_KF * 12)
    tmp32 = tMP.astype(jnp.float32).reshape(N_MP * 3)

    mesh = plsc.VectorSubcoreMesh(core_axis_name="c", subcore_axis_name="s")
    sc_call = functools.partial(
        pl.kernel,
        mesh=mesh,
        out_type=jax.ShapeDtypeStruct((M * 4,), jnp.int32),
        compiler_params=pltpu.CompilerParams(needs_layout_passes=False),
        scratch_types=[
            pltpu.VMEM((WORDS_PER_W,), jnp.int32),
            pltpu.VMEM((WORDS_PER_W,), jnp.int32),
            pltpu.VMEM((TABLE_WORDS,), jnp.int32),
            pltpu.VMEM((N_KF * 12,), jnp.float32),
            pltpu.VMEM((N_MP * 3,), jnp.float32),
            pltpu.VMEM((N_KF,), jnp.int32),
            pltpu.VMEM((N_MP,), jnp.int32),
            pltpu.VMEM((N_KF,), jnp.int32),
            pltpu.VMEM((N_MP,), jnp.int32),
        ],
    )(_sc_body)
    out_words = sc_call(meas_words, tkf32, tmp32, idxKF, idxMP)
    obs2d = lax.bitcast_convert_type(out_words.reshape(M, 2, 2), jnp.float64)
    return obs2d


# speedup vs baseline: 384.4741x; 13.8494x over previous
import functools

import jax
import jax.numpy as jnp
from jax import lax
from jax.experimental import pallas as pl
from jax.experimental.pallas import tpu as pltpu
from jax.experimental.pallas import tpu_sc as plsc

N_KF = 16
N_MP = 32
M = 65536
FX = 320.0
FY = 320.0
CX = 320.0
CY = 240.0

NUM_WORKERS = 32
MEAS_PER_W = M // NUM_WORKERS
N_TABLE = N_KF * N_MP


def _sc_body(kf_hbm, mp_hbm, tkf_hbm, tmp_hbm, idxkf_hbm, idxmp_hbm,
             x_hbm, y_hbm,
             kf_v, mp_v, x_v, y_v, tabx_v, taby_v, tkf_v, tmp_v,
             idxkf_v, idxmp_v, invkf_v, invmp_v):
    wid = lax.axis_index("s") * 2 + lax.axis_index("c")
    iota = lax.iota(jnp.int32, 16)

    pltpu.sync_copy(tkf_hbm, tkf_v)
    pltpu.sync_copy(tmp_hbm, tmp_v)
    pltpu.sync_copy(idxkf_hbm, idxkf_v)
    pltpu.sync_copy(idxmp_hbm, idxmp_v)
    pltpu.sync_copy(kf_hbm.at[pl.ds(wid * MEAS_PER_W, MEAS_PER_W)], kf_v)
    pltpu.sync_copy(mp_hbm.at[pl.ds(wid * MEAS_PER_W, MEAS_PER_W)], mp_v)

    plsc.store_scatter(invkf_v, [idxkf_v[...]], iota)
    plsc.store_scatter(invmp_v, [idxmp_v[pl.ds(0, 16)]], iota)
    plsc.store_scatter(invmp_v, [idxmp_v[pl.ds(16, 16)]], iota + 16)

    def table_step(_, cbase):
        c = cbase + iota
        kid = lax.shift_right_logical(c, jnp.int32(5))
        mid = c & 31
        kpos = plsc.load_gather(invkf_v, [kid])
        mpos = plsc.load_gather(invmp_v, [mid])
        kbase = lax.shift_left(kpos, jnp.int32(3)) + lax.shift_left(kpos, jnp.int32(2))
        mbase = mpos + lax.shift_left(mpos, jnp.int32(1))
        r = [plsc.load_gather(tkf_v, [kbase + j]) for j in range(12)]
        px = plsc.load_gather(tmp_v, [mbase])
        py = plsc.load_gather(tmp_v, [mbase + 1])
        pz = plsc.load_gather(tmp_v, [mbase + 2])
        x = r[0] * px + r[1] * py + r[2] * pz + r[3]
        y = r[4] * px + r[5] * py + r[6] * pz + r[7]
        z = r[8] * px + r[9] * py + r[10] * pz + r[11]
        inv = jnp.float32(1.0) / z
        ptx = x * inv * jnp.float32(FX) + jnp.float32(CX)
        pty = y * inv * jnp.float32(FY) + jnp.float32(CY)
        plsc.store_scatter(tabx_v, [c], ptx)
        plsc.store_scatter(taby_v, [c], pty)
        return cbase + jnp.int32(16)

    lax.fori_loop(0, N_TABLE // 16, table_step, jnp.int32(0), unroll=4)

    def gather_step(_, base):
        kf = kf_v[pl.ds(base, 16)]
        mp = mp_v[pl.ds(base, 16)]
        c = lax.shift_left(kf, jnp.int32(5)) + mp
        x_v[pl.ds(base, 16)] = plsc.load_gather(tabx_v, [c])
        y_v[pl.ds(base, 16)] = plsc.load_gather(taby_v, [c])
        return base + jnp.int32(16)

    lax.fori_loop(0, MEAS_PER_W // 16, gather_step, jnp.int32(0), unroll=8)

    pltpu.sync_copy(x_v, x_hbm.at[pl.ds(wid * MEAS_PER_W, MEAS_PER_W)])
    pltpu.sync_copy(y_v, y_hbm.at[pl.ds(wid * MEAS_PER_W, MEAS_PER_W)])


def kernel(tMP, tKF, measurements, idxMP, idxKF):
    meas32 = measurements.astype(jnp.int32)
    kf_ids = meas32[:, 0]
    mp_ids = meas32[:, 1]
    tkf32 = tKF.astype(jnp.float32)[:, :3, :].reshape(N_KF * 12)
    tmp32 = tMP.astype(jnp.float32).reshape(N_MP * 3)

    mesh = plsc.VectorSubcoreMesh(core_axis_name="c", subcore_axis_name="s")
    sc_call = functools.partial(
        pl.kernel,
        mesh=mesh,
        out_type=(
            jax.ShapeDtypeStruct((M,), jnp.float32),
            jax.ShapeDtypeStruct((M,), jnp.float32),
        ),
        compiler_params=pltpu.CompilerParams(needs_layout_passes=False),
        scratch_types=[
            pltpu.VMEM((MEAS_PER_W,), jnp.int32),
            pltpu.VMEM((MEAS_PER_W,), jnp.int32),
            pltpu.VMEM((MEAS_PER_W,), jnp.float32),
            pltpu.VMEM((MEAS_PER_W,), jnp.float32),
            pltpu.VMEM((N_TABLE,), jnp.float32),
            pltpu.VMEM((N_TABLE,), jnp.float32),
            pltpu.VMEM((N_KF * 12,), jnp.float32),
            pltpu.VMEM((N_MP * 3,), jnp.float32),
            pltpu.VMEM((N_KF,), jnp.int32),
            pltpu.VMEM((N_MP,), jnp.int32),
            pltpu.VMEM((N_KF,), jnp.int32),
            pltpu.VMEM((N_MP,), jnp.int32),
        ],
    )(_sc_body)
    out_x, out_y = sc_call(kf_ids, mp_ids, tkf32, tmp32, idxKF, idxMP)
    obs2d = jnp.stack([out_x, out_y], axis=1).astype(jnp.float64)
    return obs2d


# speedup vs baseline: 419.3461x; 1.0907x over previous
import functools

import jax
import jax.numpy as jnp
from jax import lax
from jax.experimental import pallas as pl
from jax.experimental.pallas import tpu as pltpu
from jax.experimental.pallas import tpu_sc as plsc

N_KF = 16
N_MP = 32
M = 65536
FX = 320.0
FY = 320.0
CX = 320.0
CY = 240.0

NUM_WORKERS = 32
MEAS_PER_W = M // NUM_WORKERS
N_TABLE = N_KF * N_MP


def _sc_body(kf_hbm, mp_hbm, tkf_hbm, tmp_hbm, idxkf_hbm, idxmp_hbm,
             x_hbm, y_hbm,
             kf_v, mp_v, x_v, y_v, tabx_v, taby_v, tkf_v, tmp_v,
             idxkf_v, idxmp_v, invkf_v, invmp_v,
             sem_idx, sem_tab, sem_meas):
    wid = lax.axis_index("s") * 2 + lax.axis_index("c")
    iota = lax.iota(jnp.int32, 16)

    cp_kf = pltpu.async_copy(kf_hbm.at[pl.ds(wid * MEAS_PER_W, MEAS_PER_W)], kf_v, sem_meas)
    cp_mp = pltpu.async_copy(mp_hbm.at[pl.ds(wid * MEAS_PER_W, MEAS_PER_W)], mp_v, sem_meas)
    cp_ikf = pltpu.async_copy(idxkf_hbm, idxkf_v, sem_idx)
    cp_imp = pltpu.async_copy(idxmp_hbm, idxmp_v, sem_idx)
    cp_tkf = pltpu.async_copy(tkf_hbm, tkf_v, sem_tab)
    cp_tmp = pltpu.async_copy(tmp_hbm, tmp_v, sem_tab)

    cp_ikf.wait()
    cp_imp.wait()
    plsc.store_scatter(invkf_v, [idxkf_v[...]], iota)
    plsc.store_scatter(invmp_v, [idxmp_v[pl.ds(0, 16)]], iota)
    plsc.store_scatter(invmp_v, [idxmp_v[pl.ds(16, 16)]], iota + 16)
    cp_tkf.wait()
    cp_tmp.wait()

    def table_step(_, cbase):
        c = cbase + iota
        kid = lax.shift_right_logical(c, jnp.int32(5))
        mid = c & 31
        kpos = plsc.load_gather(invkf_v, [kid])
        mpos = plsc.load_gather(invmp_v, [mid])
        kbase = lax.shift_left(kpos, jnp.int32(3)) + lax.shift_left(kpos, jnp.int32(2))
        mbase = mpos + lax.shift_left(mpos, jnp.int32(1))
        r = [plsc.load_gather(tkf_v, [kbase + j]) for j in range(12)]
        px = plsc.load_gather(tmp_v, [mbase])
        py = plsc.load_gather(tmp_v, [mbase + 1])
        pz = plsc.load_gather(tmp_v, [mbase + 2])
        x = r[0] * px + r[1] * py + r[2] * pz + r[3]
        y = r[4] * px + r[5] * py + r[6] * pz + r[7]
        z = r[8] * px + r[9] * py + r[10] * pz + r[11]
        inv = jnp.float32(1.0) / z
        ptx = x * inv * jnp.float32(FX) + jnp.float32(CX)
        pty = y * inv * jnp.float32(FY) + jnp.float32(CY)
        plsc.store_scatter(tabx_v, [c], ptx)
        plsc.store_scatter(taby_v, [c], pty)
        return cbase + jnp.int32(16)

    lax.fori_loop(0, N_TABLE // 16, table_step, jnp.int32(0), unroll=4)

    cp_kf.wait()
    cp_mp.wait()

    def gather_step(_, base):
        kf = kf_v[pl.ds(base, 16)]
        mp = mp_v[pl.ds(base, 16)]
        c = lax.shift_left(kf, jnp.int32(5)) + mp
        x_v[pl.ds(base, 16)] = plsc.load_gather(tabx_v, [c])
        y_v[pl.ds(base, 16)] = plsc.load_gather(taby_v, [c])
        return base + jnp.int32(16)

    lax.fori_loop(0, MEAS_PER_W // 16, gather_step, jnp.int32(0), unroll=8)

    pltpu.sync_copy(x_v, x_hbm.at[pl.ds(wid * MEAS_PER_W, MEAS_PER_W)])
    pltpu.sync_copy(y_v, y_hbm.at[pl.ds(wid * MEAS_PER_W, MEAS_PER_W)])


def kernel(tMP, tKF, measurements, idxMP, idxKF):
    meas32 = measurements.astype(jnp.int32)
    kf_ids = meas32[:, 0]
    mp_ids = meas32[:, 1]
    tkf32 = tKF.astype(jnp.float32)[:, :3, :].reshape(N_KF * 12)
    tmp32 = tMP.astype(jnp.float32).reshape(N_MP * 3)

    mesh = plsc.VectorSubcoreMesh(core_axis_name="c", subcore_axis_name="s")
    sc_call = functools.partial(
        pl.kernel,
        mesh=mesh,
        out_type=(
            jax.ShapeDtypeStruct((M,), jnp.float32),
            jax.ShapeDtypeStruct((M,), jnp.float32),
        ),
        compiler_params=pltpu.CompilerParams(needs_layout_passes=False),
        scratch_types=[
            pltpu.VMEM((MEAS_PER_W,), jnp.int32),
            pltpu.VMEM((MEAS_PER_W,), jnp.int32),
            pltpu.VMEM((MEAS_PER_W,), jnp.float32),
            pltpu.VMEM((MEAS_PER_W,), jnp.float32),
            pltpu.VMEM((N_TABLE,), jnp.float32),
            pltpu.VMEM((N_TABLE,), jnp.float32),
            pltpu.VMEM((N_KF * 12,), jnp.float32),
            pltpu.VMEM((N_MP * 3,), jnp.float32),
            pltpu.VMEM((N_KF,), jnp.int32),
            pltpu.VMEM((N_MP,), jnp.int32),
            pltpu.VMEM((N_KF,), jnp.int32),
            pltpu.VMEM((N_MP,), jnp.int32),
            pltpu.SemaphoreType.DMA,
            pltpu.SemaphoreType.DMA,
            pltpu.SemaphoreType.DMA,
        ],
    )(_sc_body)
    out_x, out_y = sc_call(kf_ids, mp_ids, tkf32, tmp32, idxKF, idxMP)
    obs2d = jnp.stack([out_x, out_y], axis=1).astype(jnp.float64)
    return obs2d
